# Initial kernel scaffold; baseline (speedup 1.0000x reference)
import jax
import jax.numpy as jnp
from jax.experimental import pallas as pl


def kernel(feat, edge_index, target, Ws0, Wn0, b0, Ws1, Wn1, b1, Ws2, Wn2, b2, Wc, bc):
    raise NotImplementedError("write your pallas kernel here")


# speedup vs baseline: 2.5139x; 2.5139x over previous
import jax
import jax.numpy as jnp
from jax import lax
from jax.experimental import pallas as pl
from jax.experimental.pallas import tpu as pltpu
from jax.experimental.pallas import tpu_sc as plsc

F32 = jnp.float32

NC = 2
NS = 16
NW = NC * NS
CHUNK = 128
GC = 16


def _mm_first_body(x_ref, w_ref, b_ref, z_ref, y_ref):
    h = jnp.dot(x_ref[...], w_ref[...], preferred_element_type=F32) + b_ref[...]
    z_ref[...] = h[:, :128]
    y_ref[...] = h[:, 128:]


def _mm_mid_body(z_ref, pa_ref, pb_ref, da_ref, db_ref, w_ref, b_ref,
                 z2_ref, y2_ref):
    deg = jnp.maximum(da_ref[...][:, 0:1] + db_ref[...][:, 0:1], 1.0)
    x = jnp.maximum(z_ref[...] + (pa_ref[...] + pb_ref[...]) / deg, 0.0)
    h = jnp.dot(x, w_ref[...], preferred_element_type=F32) + b_ref[...]
    z2_ref[...] = h[:, :128]
    y2_ref[...] = h[:, 128:]


def _combine_body(z_ref, pa_ref, pb_ref, da_ref, db_ref, x_ref):
    deg = jnp.maximum(da_ref[...][:, 0:1] + db_ref[...][:, 0:1], 1.0)
    x_ref[...] = jnp.maximum(z_ref[...] + (pa_ref[...] + pb_ref[...]) / deg,
                             0.0)


def _classify_body(e_ref, wc_ref, bc_ref, pred_ref, emb_ref):
    e = e_ref[...]
    nrm = jnp.sqrt(jnp.sum(e * e, axis=1, keepdims=True))
    emb = e / jnp.maximum(nrm, 1e-12)
    emb_ref[...] = emb
    pred_ref[...] = (jnp.dot(emb, wc_ref[...], preferred_element_type=F32)
                     + bc_ref[...])


def _mm_first(xp, w, b, np_, rb):
    grid = (np_ // rb,)
    return pl.pallas_call(
        _mm_first_body,
        grid=grid,
        in_specs=[
            pl.BlockSpec((rb, 128), lambda i: (i, 0)),
            pl.BlockSpec((128, 256), lambda i: (0, 0)),
            pl.BlockSpec((1, 256), lambda i: (0, 0)),
        ],
        out_specs=[pl.BlockSpec((rb, 128), lambda i: (i, 0))] * 2,
        out_shape=[jax.ShapeDtypeStruct((np_, 128), F32)] * 2,
    )(xp, w, b)


def _mm_mid(z, pa, pb, da, db, w, b, np_, rb):
    grid = (np_ // rb,)
    return pl.pallas_call(
        _mm_mid_body,
        grid=grid,
        in_specs=[
            pl.BlockSpec((rb, 128), lambda i: (i, 0)),
            pl.BlockSpec((rb, 128), lambda i: (i, 0)),
            pl.BlockSpec((rb, 128), lambda i: (i, 0)),
            pl.BlockSpec((rb, 128), lambda i: (i, 0)),
            pl.BlockSpec((rb, 128), lambda i: (i, 0)),
            pl.BlockSpec((128, 256), lambda i: (0, 0)),
            pl.BlockSpec((1, 256), lambda i: (0, 0)),
        ],
        out_specs=[pl.BlockSpec((rb, 128), lambda i: (i, 0))] * 2,
        out_shape=[jax.ShapeDtypeStruct((np_, 128), F32)] * 2,
    )(z, pa, pb, da, db, w, b)


def _combine(z, pa, pb, da, db, np_, rb):
    grid = (np_ // rb,)
    return pl.pallas_call(
        _combine_body,
        grid=grid,
        in_specs=[pl.BlockSpec((rb, 128), lambda i: (i, 0))] * 5,
        out_specs=pl.BlockSpec((rb, 128), lambda i: (i, 0)),
        out_shape=jax.ShapeDtypeStruct((np_, 128), F32),
    )(z, pa, pb, da, db)


def _classify(emb_raw, wc, bc, b_, c_):
    return pl.pallas_call(
        _classify_body,
        in_specs=[
            pl.BlockSpec((b_, 128), lambda: (0, 0)),
            pl.BlockSpec((128, c_), lambda: (0, 0)),
            pl.BlockSpec((1, c_), lambda: (0, 0)),
        ],
        out_specs=[
            pl.BlockSpec((b_, c_), lambda: (0, 0)),
            pl.BlockSpec((b_, 128), lambda: (0, 0)),
        ],
        out_shape=[
            jax.ShapeDtypeStruct((b_, c_), F32),
            jax.ShapeDtypeStruct((b_, 128), F32),
        ],
    )(emb_raw, wc, bc)


def _make_agg(np_, nchunks):
    zr = np_ // NS
    ngroups = nchunks // GC
    mesh = plsc.VectorSubcoreMesh(core_axis_name="c", subcore_axis_name="s")

    scratch = [
        pltpu.VMEM((GC, CHUNK), jnp.int32),
        pltpu.VMEM((GC, CHUNK), jnp.int32),
        pltpu.VMEM((CHUNK, 128), F32),
        pltpu.VMEM_SHARED((np_, 128), F32),
        pltpu.SemaphoreType.DMA,
    ]

    def body(y_hbm, srcb, dstb, zrow, out, src_v, dst_v, rows_v, acc, sem):
        c = lax.axis_index("c")
        s = lax.axis_index("s")
        wid = c * NS + s
        pltpu.sync_copy(zrow, acc.at[pl.ds(s * zr, zr)])
        plsc.subcore_barrier()

        def group(g, carry):
            pltpu.sync_copy(srcb.at[wid, pl.ds(g * GC, GC)], src_v)
            pltpu.sync_copy(dstb.at[wid, pl.ds(g * GC, GC)], dst_v)

            def step(r, carry2):
                pltpu.async_copy(y_hbm.at[src_v.at[r]], rows_v, sem).wait()
                pltpu.sync_copy(rows_v, acc.at[dst_v.at[r]], add=True)
                return carry2

            lax.fori_loop(0, GC, step, 0)
            return carry

        lax.fori_loop(0, ngroups, group, 0)
        plsc.subcore_barrier()
        pltpu.sync_copy(acc.at[pl.ds(s * zr, zr)],
                        out.at[c, pl.ds(s * zr, zr)])

    return pl.kernel(
        body,
        mesh=mesh,
        out_type=jax.ShapeDtypeStruct((NC, np_, 128), F32),
        scratch_types=scratch,
    )


def _make_deg(np_, nchunks):
    zr = np_ // NS
    ngroups = nchunks // GC
    mesh = plsc.VectorSubcoreMesh(core_axis_name="c", subcore_axis_name="s")

    scratch = [
        pltpu.VMEM((GC, CHUNK), jnp.int32),
        pltpu.VMEM((CHUNK, 128), F32),
        pltpu.VMEM_SHARED((np_, 128), F32),
    ]

    def body(dstb, zrow, ones_h, out, dst_v, ones_v, acc):
        c = lax.axis_index("c")
        s = lax.axis_index("s")
        wid = c * NS + s
        pltpu.sync_copy(zrow, acc.at[pl.ds(s * zr, zr)])
        pltpu.sync_copy(ones_h, ones_v)
        plsc.subcore_barrier()

        def group(g, carry):
            pltpu.sync_copy(dstb.at[wid, pl.ds(g * GC, GC)], dst_v)

            def step(r, carry2):
                pltpu.sync_copy(ones_v, acc.at[dst_v.at[r]], add=True)
                return carry2

            lax.fori_loop(0, GC, step, 0)
            return carry

        lax.fori_loop(0, ngroups, group, 0)
        plsc.subcore_barrier()
        pltpu.sync_copy(acc.at[pl.ds(s * zr, zr)],
                        out.at[c, pl.ds(s * zr, zr)])

    return pl.kernel(
        body,
        mesh=mesh,
        out_type=jax.ShapeDtypeStruct((NC, np_, 128), F32),
        scratch_types=scratch,
    )


def _make_target_gather(np_, b_):
    per = b_ // NW
    mesh = plsc.VectorSubcoreMesh(core_axis_name="c", subcore_axis_name="s")

    scratch = [
        pltpu.VMEM((per,), jnp.int32),
        pltpu.VMEM((per, 128), F32),
        pltpu.SemaphoreType.DMA,
    ]

    def body(x_hbm, tgt_hbm, out, tgt_v, rows_v, sem):
        c = lax.axis_index("c")
        s = lax.axis_index("s")
        wid = c * NS + s
        pltpu.sync_copy(tgt_hbm.at[pl.ds(wid * per, per)], tgt_v)
        pltpu.async_copy(x_hbm.at[tgt_v], rows_v, sem).wait()
        pltpu.sync_copy(rows_v, out.at[pl.ds(wid * per, per)])

    return pl.kernel(
        body,
        mesh=mesh,
        out_type=jax.ShapeDtypeStruct((b_, 128), F32),
        scratch_types=scratch,
    )


def kernel(feat, edge_index, target, Ws0, Wn0, b0, Ws1, Wn1, b1,
           Ws2, Wn2, b2, Wc, bc):
    n, d = feat.shape
    e = edge_index.shape[1]
    b_ = target.shape[0]
    c_ = Wc.shape[1]

    rb = 512
    np_ = ((n + rb - 1) // rb) * rb
    nchunks = -(-e // (NW * CHUNK))
    nchunks = ((nchunks + GC - 1) // GC) * GC
    e_pad = NW * CHUNK * nchunks

    src = edge_index[0].astype(jnp.int32)
    dst = edge_index[1].astype(jnp.int32)
    pad = e_pad - e
    srcb = jnp.concatenate([src, jnp.zeros((pad,), jnp.int32)])
    dstb = jnp.concatenate([dst, jnp.full((pad,), n, jnp.int32)])
    srcb = srcb.reshape(NW, nchunks, CHUNK)
    dstb = dstb.reshape(NW, nchunks, CHUNK)
    featp = jnp.pad(feat, ((0, np_ - n), (0, 0)))

    w0 = jnp.concatenate([Ws0, Wn0], axis=1)
    w1 = jnp.concatenate([Ws1, Wn1], axis=1)
    w2 = jnp.concatenate([Ws2, Wn2], axis=1)
    zeros_h = jnp.zeros((128,), F32)
    bf0 = jnp.concatenate([b0, zeros_h]).reshape(1, 256)
    bf1 = jnp.concatenate([b1, zeros_h]).reshape(1, 256)
    bf2 = jnp.concatenate([b2, zeros_h]).reshape(1, 256)
    bcr = bc.reshape(1, c_)

    zr = np_ // NS
    zrow = jnp.zeros((zr, 128), F32)
    ones_rows = jnp.ones((CHUNK, 128), F32)

    agg = _make_agg(np_, nchunks)
    degk = _make_deg(np_, nchunks)
    tgather = _make_target_gather(np_, b_)

    degp = degk(dstb, zrow, ones_rows)
    da, db = degp[0], degp[1]
    z0, y0 = _mm_first(featp, w0, bf0, np_, rb)
    p0 = agg(y0, srcb, dstb, zrow)
    z1, y1 = _mm_mid(z0, p0[0], p0[1], da, db, w1, bf1, np_, rb)
    p1 = agg(y1, srcb, dstb, zrow)
    z2, y2 = _mm_mid(z1, p1[0], p1[1], da, db, w2, bf2, np_, rb)
    p2 = agg(y2, srcb, dstb, zrow)
    x3 = _combine(z2, p2[0], p2[1], da, db, np_, rb)
    emb_raw = tgather(x3, target.astype(jnp.int32))
    pred, emb = _classify(emb_raw, Wc, bcr, b_, c_)
    return (pred, emb)


# speedup vs baseline: 4.1248x; 1.6408x over previous
import jax
import jax.numpy as jnp
from jax import lax
from jax.experimental import pallas as pl
from jax.experimental.pallas import tpu as pltpu
from jax.experimental.pallas import tpu_sc as plsc

F32 = jnp.float32

NC = 2
NS = 16
NW = NC * NS
CHUNK = 128
GC = 16


def _mm_first_body(x_ref, w_ref, b_ref, z_ref, y_ref):
    h = jnp.dot(x_ref[...], w_ref[...], preferred_element_type=F32) + b_ref[...]
    z_ref[...] = h[:, :128]
    y_ref[...] = h[:, 128:]


def _mm_mid_body(z_ref, pa_ref, pb_ref, da_ref, db_ref, w_ref, b_ref,
                 z2_ref, y2_ref):
    deg = jnp.maximum(da_ref[...][:, 0:1] + db_ref[...][:, 0:1], 1.0)
    x = jnp.maximum(z_ref[...] + (pa_ref[...] + pb_ref[...]) / deg, 0.0)
    h = jnp.dot(x, w_ref[...], preferred_element_type=F32) + b_ref[...]
    z2_ref[...] = h[:, :128]
    y2_ref[...] = h[:, 128:]


def _combine_body(z_ref, pa_ref, pb_ref, da_ref, db_ref, x_ref):
    deg = jnp.maximum(da_ref[...][:, 0:1] + db_ref[...][:, 0:1], 1.0)
    x_ref[...] = jnp.maximum(z_ref[...] + (pa_ref[...] + pb_ref[...]) / deg,
                             0.0)


def _classify_body(e_ref, wc_ref, bc_ref, pred_ref, emb_ref):
    e = e_ref[...]
    nrm = jnp.sqrt(jnp.sum(e * e, axis=1, keepdims=True))
    emb = e / jnp.maximum(nrm, 1e-12)
    emb_ref[...] = emb
    pred_ref[...] = (jnp.dot(emb, wc_ref[...], preferred_element_type=F32)
                     + bc_ref[...])


def _mm_first(xp, w, b, np_, rb):
    grid = (np_ // rb,)
    return pl.pallas_call(
        _mm_first_body,
        grid=grid,
        in_specs=[
            pl.BlockSpec((rb, 128), lambda i: (i, 0)),
            pl.BlockSpec((128, 256), lambda i: (0, 0)),
            pl.BlockSpec((1, 256), lambda i: (0, 0)),
        ],
        out_specs=[pl.BlockSpec((rb, 128), lambda i: (i, 0))] * 2,
        out_shape=[jax.ShapeDtypeStruct((np_, 128), F32)] * 2,
    )(xp, w, b)


def _mm_mid(z, pa, pb, da, db, w, b, np_, rb):
    grid = (np_ // rb,)
    return pl.pallas_call(
        _mm_mid_body,
        grid=grid,
        in_specs=[
            pl.BlockSpec((rb, 128), lambda i: (i, 0)),
            pl.BlockSpec((rb, 128), lambda i: (i, 0)),
            pl.BlockSpec((rb, 128), lambda i: (i, 0)),
            pl.BlockSpec((rb, 128), lambda i: (i, 0)),
            pl.BlockSpec((rb, 128), lambda i: (i, 0)),
            pl.BlockSpec((128, 256), lambda i: (0, 0)),
            pl.BlockSpec((1, 256), lambda i: (0, 0)),
        ],
        out_specs=[pl.BlockSpec((rb, 128), lambda i: (i, 0))] * 2,
        out_shape=[jax.ShapeDtypeStruct((np_, 128), F32)] * 2,
    )(z, pa, pb, da, db, w, b)


def _combine(z, pa, pb, da, db, np_, rb):
    grid = (np_ // rb,)
    return pl.pallas_call(
        _combine_body,
        grid=grid,
        in_specs=[pl.BlockSpec((rb, 128), lambda i: (i, 0))] * 5,
        out_specs=pl.BlockSpec((rb, 128), lambda i: (i, 0)),
        out_shape=jax.ShapeDtypeStruct((np_, 128), F32),
    )(z, pa, pb, da, db)


def _classify(emb_raw, wc, bc, b_, c_):
    return pl.pallas_call(
        _classify_body,
        in_specs=[
            pl.BlockSpec((b_, 128), lambda: (0, 0)),
            pl.BlockSpec((128, c_), lambda: (0, 0)),
            pl.BlockSpec((1, c_), lambda: (0, 0)),
        ],
        out_specs=[
            pl.BlockSpec((b_, c_), lambda: (0, 0)),
            pl.BlockSpec((b_, 128), lambda: (0, 0)),
        ],
        out_shape=[
            jax.ShapeDtypeStruct((b_, c_), F32),
            jax.ShapeDtypeStruct((b_, 128), F32),
        ],
    )(emb_raw, wc, bc)


def _make_agg(np_, nchunks):
    zr = np_ // NS
    mesh = plsc.VectorSubcoreMesh(core_axis_name="c", subcore_axis_name="s")

    scratch = [
        pltpu.VMEM((2, CHUNK), jnp.int32),
        pltpu.VMEM((2, CHUNK), jnp.int32),
        pltpu.VMEM((2, CHUNK, 128), F32),
        pltpu.VMEM_SHARED((np_, 128), F32),
        pltpu.SemaphoreType.DMA((2,)),
    ]

    def body(y_hbm, srcb, dstb, zrow, out, src_v, dst_v, rows_v, acc, sems):
        c = lax.axis_index("c")
        s = lax.axis_index("s")
        wid = c * NS + s
        pltpu.sync_copy(zrow, acc.at[pl.ds(s * zr, zr)])
        plsc.subcore_barrier()

        pltpu.sync_copy(srcb.at[wid, 0], src_v.at[0])
        pltpu.sync_copy(dstb.at[wid, 0], dst_v.at[0])
        pltpu.async_copy(y_hbm.at[src_v.at[0]], rows_v.at[0], sems.at[0])

        def step(r, carry):
            bi = lax.rem(r, 2)
            nb = lax.rem(r + 1, 2)

            @pl.when(r + 1 < nchunks)
            def _():
                pltpu.sync_copy(srcb.at[wid, r + 1], src_v.at[nb])
                pltpu.sync_copy(dstb.at[wid, r + 1], dst_v.at[nb])
                pltpu.async_copy(y_hbm.at[src_v.at[nb]], rows_v.at[nb],
                                 sems.at[nb])

            pltpu.make_async_copy(y_hbm.at[src_v.at[bi]], rows_v.at[bi],
                                  sems.at[bi]).wait()
            pltpu.sync_copy(rows_v.at[bi], acc.at[dst_v.at[bi]], add=True)
            return carry

        lax.fori_loop(0, nchunks, step, 0)
        plsc.subcore_barrier()
        pltpu.sync_copy(acc.at[pl.ds(s * zr, zr)],
                        out.at[c, pl.ds(s * zr, zr)])

    return pl.kernel(
        body,
        mesh=mesh,
        out_type=jax.ShapeDtypeStruct((NC, np_, 128), F32),
        scratch_types=scratch,
    )


def _make_deg(np_, nchunks):
    zr = np_ // NS
    mesh = plsc.VectorSubcoreMesh(core_axis_name="c", subcore_axis_name="s")

    scratch = [
        pltpu.VMEM((2, CHUNK), jnp.int32),
        pltpu.VMEM((CHUNK, 128), F32),
        pltpu.VMEM_SHARED((np_, 128), F32),
    ]

    def body(dstb, zrow, ones_h, out, dst_v, ones_v, acc):
        c = lax.axis_index("c")
        s = lax.axis_index("s")
        wid = c * NS + s
        pltpu.sync_copy(zrow, acc.at[pl.ds(s * zr, zr)])
        pltpu.sync_copy(ones_h, ones_v)
        plsc.subcore_barrier()

        pltpu.sync_copy(dstb.at[wid, 0], dst_v.at[0])

        def step(r, carry):
            bi = lax.rem(r, 2)
            nb = lax.rem(r + 1, 2)

            @pl.when(r + 1 < nchunks)
            def _():
                pltpu.sync_copy(dstb.at[wid, r + 1], dst_v.at[nb])

            pltpu.sync_copy(ones_v, acc.at[dst_v.at[bi]], add=True)
            return carry

        lax.fori_loop(0, nchunks, step, 0)
        plsc.subcore_barrier()
        pltpu.sync_copy(acc.at[pl.ds(s * zr, zr)],
                        out.at[c, pl.ds(s * zr, zr)])

    return pl.kernel(
        body,
        mesh=mesh,
        out_type=jax.ShapeDtypeStruct((NC, np_, 128), F32),
        scratch_types=scratch,
    )


def _make_target_gather(np_, b_):
    per = b_ // NW
    mesh = plsc.VectorSubcoreMesh(core_axis_name="c", subcore_axis_name="s")

    scratch = [
        pltpu.VMEM((per,), jnp.int32),
        pltpu.VMEM((per, 128), F32),
        pltpu.SemaphoreType.DMA,
    ]

    def body(x_hbm, tgt_hbm, out, tgt_v, rows_v, sem):
        c = lax.axis_index("c")
        s = lax.axis_index("s")
        wid = c * NS + s
        pltpu.sync_copy(tgt_hbm.at[pl.ds(wid * per, per)], tgt_v)
        pltpu.async_copy(x_hbm.at[tgt_v], rows_v, sem).wait()
        pltpu.sync_copy(rows_v, out.at[pl.ds(wid * per, per)])

    return pl.kernel(
        body,
        mesh=mesh,
        out_type=jax.ShapeDtypeStruct((b_, 128), F32),
        scratch_types=scratch,
    )


def kernel(feat, edge_index, target, Ws0, Wn0, b0, Ws1, Wn1, b1,
           Ws2, Wn2, b2, Wc, bc):
    n, d = feat.shape
    e = edge_index.shape[1]
    b_ = target.shape[0]
    c_ = Wc.shape[1]

    rb = 512
    np_ = ((n + rb - 1) // rb) * rb
    if np_ == n:
        np_ += rb
    nchunks = -(-e // (NW * CHUNK))
    e_pad = NW * CHUNK * nchunks

    src = edge_index[0].astype(jnp.int32)
    dst = edge_index[1].astype(jnp.int32)
    pad = e_pad - e
    srcb = jnp.concatenate([src, jnp.zeros((pad,), jnp.int32)])
    dummy = n + (jnp.arange(pad, dtype=jnp.int32) % jnp.int32(np_ - n))
    dstb = jnp.concatenate([dst, dummy])
    srcb = srcb.reshape(NW, nchunks, CHUNK)
    dstb = dstb.reshape(NW, nchunks, CHUNK)
    featp = jnp.pad(feat, ((0, np_ - n), (0, 0)))

    w0 = jnp.concatenate([Ws0, Wn0], axis=1)
    w1 = jnp.concatenate([Ws1, Wn1], axis=1)
    w2 = jnp.concatenate([Ws2, Wn2], axis=1)
    zeros_h = jnp.zeros((128,), F32)
    bf0 = jnp.concatenate([b0, zeros_h]).reshape(1, 256)
    bf1 = jnp.concatenate([b1, zeros_h]).reshape(1, 256)
    bf2 = jnp.concatenate([b2, zeros_h]).reshape(1, 256)
    bcr = bc.reshape(1, c_)

    zr = np_ // NS
    zrow = jnp.zeros((zr, 128), F32)
    ones_rows = jnp.ones((CHUNK, 128), F32)

    agg = _make_agg(np_, nchunks)
    degk = _make_deg(np_, nchunks)
    tgather = _make_target_gather(np_, b_)

    degp = degk(dstb, zrow, ones_rows)
    da, db = degp[0], degp[1]
    z0, y0 = _mm_first(featp, w0, bf0, np_, rb)
    p0 = agg(y0, srcb, dstb, zrow)
    z1, y1 = _mm_mid(z0, p0[0], p0[1], da, db, w1, bf1, np_, rb)
    p1 = agg(y1, srcb, dstb, zrow)
    z2, y2 = _mm_mid(z1, p1[0], p1[1], da, db, w2, bf2, np_, rb)
    p2 = agg(y2, srcb, dstb, zrow)
    x3 = _combine(z2, p2[0], p2[1], da, db, np_, rb)
    emb_raw = tgather(x3, target.astype(jnp.int32))
    pred, emb = _classify(emb_raw, Wc, bcr, b_, c_)
    return (pred, emb)


# speedup vs baseline: 6.9659x; 1.6888x over previous
import jax
import jax.numpy as jnp
from jax import lax
from jax.experimental import pallas as pl
from jax.experimental.pallas import tpu as pltpu
from jax.experimental.pallas import tpu_sc as plsc

F32 = jnp.float32

NC = 2
NS = 16
NW = NC * NS
CHUNK = 128
GC = 16


def _mm_first_body(x_ref, w_ref, b_ref, z_ref, y_ref):
    h = jnp.dot(x_ref[...], w_ref[...], preferred_element_type=F32) + b_ref[...]
    z_ref[...] = h[:, :128]
    y_ref[...] = h[:, 128:]


def _mm_mid_body(z_ref, pa_ref, pb_ref, da_ref, db_ref, w_ref, b_ref,
                 z2_ref, y2_ref):
    deg = jnp.maximum(da_ref[...][:, 0:1] + db_ref[...][:, 0:1], 1.0)
    x = jnp.maximum(z_ref[...] + (pa_ref[...] + pb_ref[...]) / deg, 0.0)
    h = jnp.dot(x, w_ref[...], preferred_element_type=F32) + b_ref[...]
    z2_ref[...] = h[:, :128]
    y2_ref[...] = h[:, 128:]


def _combine_body(z_ref, pa_ref, pb_ref, da_ref, db_ref, x_ref):
    deg = jnp.maximum(da_ref[...][:, 0:1] + db_ref[...][:, 0:1], 1.0)
    x_ref[...] = jnp.maximum(z_ref[...] + (pa_ref[...] + pb_ref[...]) / deg,
                             0.0)


def _classify_body(e_ref, wc_ref, bc_ref, pred_ref, emb_ref):
    e = e_ref[...]
    nrm = jnp.sqrt(jnp.sum(e * e, axis=1, keepdims=True))
    emb = e / jnp.maximum(nrm, 1e-12)
    emb_ref[...] = emb
    pred_ref[...] = (jnp.dot(emb, wc_ref[...], preferred_element_type=F32)
                     + bc_ref[...])


def _mm_first(xp, w, b, np_, rb):
    grid = (np_ // rb,)
    return pl.pallas_call(
        _mm_first_body,
        grid=grid,
        in_specs=[
            pl.BlockSpec((rb, 128), lambda i: (i, 0)),
            pl.BlockSpec((128, 256), lambda i: (0, 0)),
            pl.BlockSpec((1, 256), lambda i: (0, 0)),
        ],
        out_specs=[pl.BlockSpec((rb, 128), lambda i: (i, 0))] * 2,
        out_shape=[jax.ShapeDtypeStruct((np_, 128), F32)] * 2,
    )(xp, w, b)


def _mm_mid(z, pa, pb, da, db, w, b, np_, rb):
    grid = (np_ // rb,)
    return pl.pallas_call(
        _mm_mid_body,
        grid=grid,
        in_specs=[
            pl.BlockSpec((rb, 128), lambda i: (i, 0)),
            pl.BlockSpec((rb, 128), lambda i: (i, 0)),
            pl.BlockSpec((rb, 128), lambda i: (i, 0)),
            pl.BlockSpec((rb, 128), lambda i: (i, 0)),
            pl.BlockSpec((rb, 128), lambda i: (i, 0)),
            pl.BlockSpec((128, 256), lambda i: (0, 0)),
            pl.BlockSpec((1, 256), lambda i: (0, 0)),
        ],
        out_specs=[pl.BlockSpec((rb, 128), lambda i: (i, 0))] * 2,
        out_shape=[jax.ShapeDtypeStruct((np_, 128), F32)] * 2,
    )(z, pa, pb, da, db, w, b)


def _combine(z, pa, pb, da, db, np_, rb):
    grid = (np_ // rb,)
    return pl.pallas_call(
        _combine_body,
        grid=grid,
        in_specs=[pl.BlockSpec((rb, 128), lambda i: (i, 0))] * 5,
        out_specs=pl.BlockSpec((rb, 128), lambda i: (i, 0)),
        out_shape=jax.ShapeDtypeStruct((np_, 128), F32),
    )(z, pa, pb, da, db)


def _classify(emb_raw, wc, bc, b_, c_):
    return pl.pallas_call(
        _classify_body,
        in_specs=[
            pl.BlockSpec((b_, 128), lambda: (0, 0)),
            pl.BlockSpec((128, c_), lambda: (0, 0)),
            pl.BlockSpec((1, c_), lambda: (0, 0)),
        ],
        out_specs=[
            pl.BlockSpec((b_, c_), lambda: (0, 0)),
            pl.BlockSpec((b_, 128), lambda: (0, 0)),
        ],
        out_shape=[
            jax.ShapeDtypeStruct((b_, c_), F32),
            jax.ShapeDtypeStruct((b_, 128), F32),
        ],
    )(emb_raw, wc, bc)


def _make_agg(np_, nchunks):
    zr = np_ // NS
    mesh = plsc.VectorSubcoreMesh(core_axis_name="c", subcore_axis_name="s")

    scratch = [
        pltpu.VMEM((2, CHUNK), jnp.int32),
        pltpu.VMEM((2, CHUNK), jnp.int32),
        pltpu.VMEM((2, CHUNK, 128), F32),
        pltpu.VMEM_SHARED((np_, 128), F32),
        pltpu.SemaphoreType.DMA((2,)),
    ]

    def body(y_hbm, srcb, dstb, zrow, out, src_v, dst_v, rows_v, acc, sems):
        c = lax.axis_index("c")
        s = lax.axis_index("s")
        wid = c * NS + s
        pltpu.sync_copy(zrow, acc.at[pl.ds(s * zr, zr)])
        plsc.subcore_barrier()

        pltpu.sync_copy(srcb.at[wid, 0], src_v.at[0])
        pltpu.sync_copy(dstb.at[wid, 0], dst_v.at[0])
        pltpu.async_copy(y_hbm.at[src_v.at[0]], rows_v.at[0], sems.at[0])

        def step(r, carry):
            bi = lax.rem(r, 2)
            nb = lax.rem(r + 1, 2)

            @pl.when(r + 1 < nchunks)
            def _():
                pltpu.sync_copy(srcb.at[wid, r + 1], src_v.at[nb])
                pltpu.sync_copy(dstb.at[wid, r + 1], dst_v.at[nb])
                pltpu.async_copy(y_hbm.at[src_v.at[nb]], rows_v.at[nb],
                                 sems.at[nb])

            pltpu.make_async_copy(y_hbm.at[src_v.at[bi]], rows_v.at[bi],
                                  sems.at[bi]).wait()
            pltpu.sync_copy(rows_v.at[bi], acc.at[dst_v.at[bi]], add=True)
            return carry

        lax.fori_loop(0, nchunks, step, 0)
        plsc.subcore_barrier()
        pltpu.sync_copy(acc.at[pl.ds(s * zr, zr)],
                        out.at[c, pl.ds(s * zr, zr)])

    return pl.kernel(
        body,
        mesh=mesh,
        out_type=jax.ShapeDtypeStruct((NC, np_, 128), F32),
        scratch_types=scratch,
    )


def _make_deg(np_, nchunks):
    zr = np_ // NS
    mesh = plsc.VectorSubcoreMesh(core_axis_name="c", subcore_axis_name="s")

    scratch = [
        pltpu.VMEM((2, CHUNK), jnp.int32),
        pltpu.VMEM((CHUNK, 128), F32),
        pltpu.VMEM_SHARED((np_, 128), F32),
    ]

    def body(dstb, zrow, ones_h, out, dst_v, ones_v, acc):
        c = lax.axis_index("c")
        s = lax.axis_index("s")
        wid = c * NS + s
        pltpu.sync_copy(zrow, acc.at[pl.ds(s * zr, zr)])
        pltpu.sync_copy(ones_h, ones_v)
        plsc.subcore_barrier()

        pltpu.sync_copy(dstb.at[wid, 0], dst_v.at[0])

        def step(r, carry):
            bi = lax.rem(r, 2)
            nb = lax.rem(r + 1, 2)

            @pl.when(r + 1 < nchunks)
            def _():
                pltpu.sync_copy(dstb.at[wid, r + 1], dst_v.at[nb])

            pltpu.sync_copy(ones_v, acc.at[dst_v.at[bi]], add=True)
            return carry

        lax.fori_loop(0, nchunks, step, 0)
        plsc.subcore_barrier()
        pltpu.sync_copy(acc.at[pl.ds(s * zr, zr)],
                        out.at[c, pl.ds(s * zr, zr)])

    return pl.kernel(
        body,
        mesh=mesh,
        out_type=jax.ShapeDtypeStruct((NC, np_, 128), F32),
        scratch_types=scratch,
    )


def _make_target_gather(np_, b_):
    per = b_ // NW
    mesh = plsc.VectorSubcoreMesh(core_axis_name="c", subcore_axis_name="s")

    scratch = [
        pltpu.VMEM((per,), jnp.int32),
        pltpu.VMEM((per, 128), F32),
        pltpu.SemaphoreType.DMA,
    ]

    def body(x_hbm, tgt_hbm, out, tgt_v, rows_v, sem):
        c = lax.axis_index("c")
        s = lax.axis_index("s")
        wid = c * NS + s
        pltpu.sync_copy(tgt_hbm.at[pl.ds(wid * per, per)], tgt_v)
        pltpu.async_copy(x_hbm.at[tgt_v], rows_v, sem).wait()
        pltpu.sync_copy(rows_v, out.at[pl.ds(wid * per, per)])

    return pl.kernel(
        body,
        mesh=mesh,
        out_type=jax.ShapeDtypeStruct((b_, 128), F32),
        scratch_types=scratch,
    )


def kernel(feat, edge_index, target, Ws0, Wn0, b0, Ws1, Wn1, b1,
           Ws2, Wn2, b2, Wc, bc):
    n, d = feat.shape
    e = edge_index.shape[1]
    b_ = target.shape[0]
    c_ = Wc.shape[1]

    rb = 512
    np_ = ((n + rb - 1) // rb) * rb
    if np_ == n:
        np_ += rb
    nchunks = -(-e // (NW * CHUNK))
    e_pad = NW * CHUNK * nchunks

    src = edge_index[0].astype(jnp.int32)
    dst = edge_index[1].astype(jnp.int32)
    pad = e_pad - e
    dummy_src = jnp.arange(pad, dtype=jnp.int32) % jnp.int32(n)
    srcb = jnp.concatenate([src, dummy_src])
    dummy = n + (jnp.arange(pad, dtype=jnp.int32) % jnp.int32(np_ - n))
    dstb = jnp.concatenate([dst, dummy])
    srcb = srcb.reshape(NW, nchunks, CHUNK)
    dstb = dstb.reshape(NW, nchunks, CHUNK)
    featp = jnp.pad(feat, ((0, np_ - n), (0, 0)))

    w0 = jnp.concatenate([Ws0, Wn0], axis=1)
    w1 = jnp.concatenate([Ws1, Wn1], axis=1)
    w2 = jnp.concatenate([Ws2, Wn2], axis=1)
    zeros_h = jnp.zeros((128,), F32)
    bf0 = jnp.concatenate([b0, zeros_h]).reshape(1, 256)
    bf1 = jnp.concatenate([b1, zeros_h]).reshape(1, 256)
    bf2 = jnp.concatenate([b2, zeros_h]).reshape(1, 256)
    bcr = bc.reshape(1, c_)

    zr = np_ // NS
    zrow = jnp.zeros((zr, 128), F32)
    ones_rows = jnp.ones((CHUNK, 128), F32)

    agg = _make_agg(np_, nchunks)
    degk = _make_deg(np_, nchunks)
    tgather = _make_target_gather(np_, b_)

    degp = degk(dstb, zrow, ones_rows)
    da, db = degp[0], degp[1]
    z0, y0 = _mm_first(featp, w0, bf0, np_, rb)
    p0 = agg(y0, srcb, dstb, zrow)
    z1, y1 = _mm_mid(z0, p0[0], p0[1], da, db, w1, bf1, np_, rb)
    p1 = agg(y1, srcb, dstb, zrow)
    z2, y2 = _mm_mid(z1, p1[0], p1[1], da, db, w2, bf2, np_, rb)
    p2 = agg(y2, srcb, dstb, zrow)
    x3 = _combine(z2, p2[0], p2[1], da, db, np_, rb)
    emb_raw = tgather(x3, target.astype(jnp.int32))
    pred, emb = _classify(emb_raw, Wc, bcr, b_, c_)
    return (pred, emb)


# speedup vs baseline: 7.2358x; 1.0388x over previous
import jax
import jax.numpy as jnp
from jax import lax
from jax.experimental import pallas as pl
from jax.experimental.pallas import tpu as pltpu
from jax.experimental.pallas import tpu_sc as plsc

F32 = jnp.float32

NC = 2
NS = 16
NW = NC * NS
CHUNK = 128
GC = 16


def _mm_first_body(x_ref, w_ref, b_ref, z_ref, y_ref):
    h = jnp.dot(x_ref[...], w_ref[...], preferred_element_type=F32) + b_ref[...]
    z_ref[...] = h[:, :128]
    y_ref[...] = h[:, 128:]


def _invdeg_body(da_ref, db_ref, iv_ref):
    iv_ref[...] = 1.0 / jnp.maximum(
        da_ref[...][:, 0:1] + db_ref[...][:, 0:1], 1.0)


def _mm_mid_body(z_ref, pa_ref, pb_ref, iv_ref, w_ref, b_ref,
                 z2_ref, y2_ref):
    x = jnp.maximum(z_ref[...] + (pa_ref[...] + pb_ref[...]) * iv_ref[...],
                    0.0)
    h = jnp.dot(x, w_ref[...], preferred_element_type=F32) + b_ref[...]
    z2_ref[...] = h[:, :128]
    y2_ref[...] = h[:, 128:]


def _classify_body(z_ref, pa_ref, pb_ref, da_ref, db_ref, wc_ref, bc_ref,
                   pred_ref, emb_ref):
    deg = jnp.maximum(da_ref[...][:, 0:1] + db_ref[...][:, 0:1], 1.0)
    e = jnp.maximum(z_ref[...] + (pa_ref[...] + pb_ref[...]) / deg, 0.0)
    nrm = jnp.sqrt(jnp.sum(e * e, axis=1, keepdims=True))
    emb = e / jnp.maximum(nrm, 1e-12)
    emb_ref[...] = emb
    pred_ref[...] = (jnp.dot(emb, wc_ref[...], preferred_element_type=F32)
                     + bc_ref[...])


def _mm_first(xp, w, b, np_, rb):
    grid = (np_ // rb,)
    return pl.pallas_call(
        _mm_first_body,
        grid=grid,
        in_specs=[
            pl.BlockSpec((rb, 128), lambda i: (i, 0)),
            pl.BlockSpec((128, 256), lambda i: (0, 0)),
            pl.BlockSpec((1, 256), lambda i: (0, 0)),
        ],
        out_specs=[pl.BlockSpec((rb, 128), lambda i: (i, 0))] * 2,
        out_shape=[jax.ShapeDtypeStruct((np_, 128), F32)] * 2,
    )(xp, w, b)


def _invdeg(da, db, np_, rb):
    grid = (np_ // rb,)
    return pl.pallas_call(
        _invdeg_body,
        grid=grid,
        in_specs=[pl.BlockSpec((rb, 128), lambda i: (i, 0))] * 2,
        out_specs=pl.BlockSpec((rb, 1), lambda i: (i, 0)),
        out_shape=jax.ShapeDtypeStruct((np_, 1), F32),
    )(da, db)


def _mm_mid(z, pa, pb, iv, w, b, np_, rb):
    grid = (np_ // rb,)
    return pl.pallas_call(
        _mm_mid_body,
        grid=grid,
        in_specs=[
            pl.BlockSpec((rb, 128), lambda i: (i, 0)),
            pl.BlockSpec((rb, 128), lambda i: (i, 0)),
            pl.BlockSpec((rb, 128), lambda i: (i, 0)),
            pl.BlockSpec((rb, 1), lambda i: (i, 0)),
            pl.BlockSpec((128, 256), lambda i: (0, 0)),
            pl.BlockSpec((1, 256), lambda i: (0, 0)),
        ],
        out_specs=[pl.BlockSpec((rb, 128), lambda i: (i, 0))] * 2,
        out_shape=[jax.ShapeDtypeStruct((np_, 128), F32)] * 2,
    )(z, pa, pb, iv, w, b)


def _classify(zt, pat, pbt, dat, dbt, wc, bc, b_, c_):
    return pl.pallas_call(
        _classify_body,
        in_specs=[
            pl.BlockSpec((b_, 128), lambda: (0, 0)),
            pl.BlockSpec((b_, 128), lambda: (0, 0)),
            pl.BlockSpec((b_, 128), lambda: (0, 0)),
            pl.BlockSpec((b_, 128), lambda: (0, 0)),
            pl.BlockSpec((b_, 128), lambda: (0, 0)),
            pl.BlockSpec((128, c_), lambda: (0, 0)),
            pl.BlockSpec((1, c_), lambda: (0, 0)),
        ],
        out_specs=[
            pl.BlockSpec((b_, c_), lambda: (0, 0)),
            pl.BlockSpec((b_, 128), lambda: (0, 0)),
        ],
        out_shape=[
            jax.ShapeDtypeStruct((b_, c_), F32),
            jax.ShapeDtypeStruct((b_, 128), F32),
        ],
    )(zt, pat, pbt, dat, dbt, wc, bc)


def _make_agg(np_, nchunks):
    zr = np_ // NS
    mesh = plsc.VectorSubcoreMesh(core_axis_name="c", subcore_axis_name="s")

    scratch = [
        pltpu.VMEM((2, CHUNK), jnp.int32),
        pltpu.VMEM((2, CHUNK), jnp.int32),
        pltpu.VMEM((2, CHUNK, 128), F32),
        pltpu.VMEM_SHARED((np_, 128), F32),
        pltpu.SemaphoreType.DMA((2,)),
    ]

    def body(y_hbm, srcb, dstb, zrow, out, src_v, dst_v, rows_v, acc, sems):
        c = lax.axis_index("c")
        s = lax.axis_index("s")
        wid = c * NS + s
        pltpu.sync_copy(zrow, acc.at[pl.ds(s * zr, zr)])
        plsc.subcore_barrier()

        pltpu.sync_copy(srcb.at[wid, 0], src_v.at[0])
        pltpu.sync_copy(dstb.at[wid, 0], dst_v.at[0])
        pltpu.async_copy(y_hbm.at[src_v.at[0]], rows_v.at[0], sems.at[0])

        def step(r, carry):
            bi = lax.rem(r, 2)
            nb = lax.rem(r + 1, 2)

            @pl.when(r + 1 < nchunks)
            def _():
                pltpu.sync_copy(srcb.at[wid, r + 1], src_v.at[nb])
                pltpu.sync_copy(dstb.at[wid, r + 1], dst_v.at[nb])
                pltpu.async_copy(y_hbm.at[src_v.at[nb]], rows_v.at[nb],
                                 sems.at[nb])

            pltpu.make_async_copy(y_hbm.at[src_v.at[bi]], rows_v.at[bi],
                                  sems.at[bi]).wait()
            pltpu.sync_copy(rows_v.at[bi], acc.at[dst_v.at[bi]], add=True)
            return carry

        lax.fori_loop(0, nchunks, step, 0)
        plsc.subcore_barrier()
        pltpu.sync_copy(acc.at[pl.ds(s * zr, zr)],
                        out.at[c, pl.ds(s * zr, zr)])

    return pl.kernel(
        body,
        mesh=mesh,
        out_type=jax.ShapeDtypeStruct((NC, np_, 128), F32),
        scratch_types=scratch,
    )


def _make_deg(np_, nchunks):
    zr = np_ // NS
    mesh = plsc.VectorSubcoreMesh(core_axis_name="c", subcore_axis_name="s")

    scratch = [
        pltpu.VMEM((2, CHUNK), jnp.int32),
        pltpu.VMEM((CHUNK, 128), F32),
        pltpu.VMEM_SHARED((np_, 128), F32),
    ]

    def body(dstb, zrow, ones_h, out, dst_v, ones_v, acc):
        c = lax.axis_index("c")
        s = lax.axis_index("s")
        wid = c * NS + s
        pltpu.sync_copy(zrow, acc.at[pl.ds(s * zr, zr)])
        pltpu.sync_copy(ones_h, ones_v)
        plsc.subcore_barrier()

        pltpu.sync_copy(dstb.at[wid, 0], dst_v.at[0])

        def step(r, carry):
            bi = lax.rem(r, 2)
            nb = lax.rem(r + 1, 2)

            @pl.when(r + 1 < nchunks)
            def _():
                pltpu.sync_copy(dstb.at[wid, r + 1], dst_v.at[nb])

            pltpu.sync_copy(ones_v, acc.at[dst_v.at[bi]], add=True)
            return carry

        lax.fori_loop(0, nchunks, step, 0)
        plsc.subcore_barrier()
        pltpu.sync_copy(acc.at[pl.ds(s * zr, zr)],
                        out.at[c, pl.ds(s * zr, zr)])

    return pl.kernel(
        body,
        mesh=mesh,
        out_type=jax.ShapeDtypeStruct((NC, np_, 128), F32),
        scratch_types=scratch,
    )


def _make_target_gather(np_, b_, ntab):
    per = b_ // NW
    mesh = plsc.VectorSubcoreMesh(core_axis_name="c", subcore_axis_name="s")

    scratch = [
        pltpu.VMEM((per,), jnp.int32),
        pltpu.VMEM((per, 128), F32),
        pltpu.SemaphoreType.DMA,
    ]

    def body(*refs):
        tables = refs[:ntab]
        tgt_hbm, out, tgt_v, rows_v, sem = refs[ntab:]
        c = lax.axis_index("c")
        s = lax.axis_index("s")
        wid = c * NS + s
        pltpu.sync_copy(tgt_hbm.at[pl.ds(wid * per, per)], tgt_v)
        for k, tab in enumerate(tables):
            pltpu.async_copy(tab.at[tgt_v], rows_v, sem).wait()
            pltpu.sync_copy(rows_v, out.at[k, pl.ds(wid * per, per)])

    return pl.kernel(
        body,
        mesh=mesh,
        out_type=jax.ShapeDtypeStruct((ntab, b_, 128), F32),
        scratch_types=scratch,
    )


def kernel(feat, edge_index, target, Ws0, Wn0, b0, Ws1, Wn1, b1,
           Ws2, Wn2, b2, Wc, bc):
    n, d = feat.shape
    e = edge_index.shape[1]
    b_ = target.shape[0]
    c_ = Wc.shape[1]

    rb = 512
    np_ = ((n + rb - 1) // rb) * rb
    if np_ == n:
        np_ += rb
    nchunks = -(-e // (NW * CHUNK))
    e_pad = NW * CHUNK * nchunks

    src = edge_index[0].astype(jnp.int32)
    dst = edge_index[1].astype(jnp.int32)
    pad = e_pad - e
    dummy_src = jnp.arange(pad, dtype=jnp.int32) % jnp.int32(n)
    srcb = jnp.concatenate([src, dummy_src])
    dummy = n + (jnp.arange(pad, dtype=jnp.int32) % jnp.int32(np_ - n))
    dstb = jnp.concatenate([dst, dummy])
    srcb = srcb.reshape(NW, nchunks, CHUNK)
    dstb = dstb.reshape(NW, nchunks, CHUNK)
    featp = jnp.pad(feat, ((0, np_ - n), (0, 0)))

    w0 = jnp.concatenate([Ws0, Wn0], axis=1)
    w1 = jnp.concatenate([Ws1, Wn1], axis=1)
    w2 = jnp.concatenate([Ws2, Wn2], axis=1)
    zeros_h = jnp.zeros((128,), F32)
    bf0 = jnp.concatenate([b0, zeros_h]).reshape(1, 256)
    bf1 = jnp.concatenate([b1, zeros_h]).reshape(1, 256)
    bf2 = jnp.concatenate([b2, zeros_h]).reshape(1, 256)
    bcr = bc.reshape(1, c_)

    zr = np_ // NS
    zrow = jnp.zeros((zr, 128), F32)
    ones_rows = jnp.ones((CHUNK, 128), F32)

    agg = _make_agg(np_, nchunks)
    degk = _make_deg(np_, nchunks)
    tgather = _make_target_gather(np_, b_, 5)

    degp = degk(dstb, zrow, ones_rows)
    da, db = degp[0], degp[1]
    iv = _invdeg(da, db, np_, rb)
    z0, y0 = _mm_first(featp, w0, bf0, np_, rb)
    p0 = agg(y0, srcb, dstb, zrow)
    z1, y1 = _mm_mid(z0, p0[0], p0[1], iv, w1, bf1, np_, rb)
    p1 = agg(y1, srcb, dstb, zrow)
    z2, y2 = _mm_mid(z1, p1[0], p1[1], iv, w2, bf2, np_, rb)
    p2 = agg(y2, srcb, dstb, zrow)
    g = tgather(z2, p2[0], p2[1], da, db, target.astype(jnp.int32))
    pred, emb = _classify(g[0], g[1], g[2], g[3], g[4], Wc, bcr, b_, c_)
    return (pred, emb)


# speedup vs baseline: 7.2433x; 1.0010x over previous
import jax
import jax.numpy as jnp
from jax import lax
from jax.experimental import pallas as pl
from jax.experimental.pallas import tpu as pltpu
from jax.experimental.pallas import tpu_sc as plsc

F32 = jnp.float32

NC = 2
NS = 16
NW = NC * NS
CHUNK = 128
GC = 16


def _mm_first_body(x_ref, w_ref, b_ref, z_ref, y_ref):
    h = jnp.dot(x_ref[...], w_ref[...], preferred_element_type=F32) + b_ref[...]
    z_ref[...] = h[:, :128]
    y_ref[...] = h[:, 128:]


def _invdeg_body(da_ref, db_ref, iv_ref):
    iv_ref[...] = 1.0 / jnp.maximum(
        da_ref[...][:, 0:1] + db_ref[...][:, 0:1], 1.0)


def _mm_mid_body(z_ref, pa_ref, pb_ref, iv_ref, w_ref, b_ref,
                 z2_ref, y2_ref):
    x = jnp.maximum(z_ref[...] + (pa_ref[...] + pb_ref[...]) * iv_ref[...],
                    0.0)
    h = jnp.dot(x, w_ref[...], preferred_element_type=F32) + b_ref[...]
    z2_ref[...] = h[:, :128]
    y2_ref[...] = h[:, 128:]


def _classify_body(z_ref, pa_ref, pb_ref, da_ref, db_ref, wc_ref, bc_ref,
                   pred_ref, emb_ref):
    deg = jnp.maximum(da_ref[...][:, 0:1] + db_ref[...][:, 0:1], 1.0)
    e = jnp.maximum(z_ref[...] + (pa_ref[...] + pb_ref[...]) / deg, 0.0)
    nrm = jnp.sqrt(jnp.sum(e * e, axis=1, keepdims=True))
    emb = e / jnp.maximum(nrm, 1e-12)
    emb_ref[...] = emb
    pred_ref[...] = (jnp.dot(emb, wc_ref[...], preferred_element_type=F32)
                     + bc_ref[...])


def _mm_first(xp, w, b, np_, rb):
    grid = (np_ // rb,)
    return pl.pallas_call(
        _mm_first_body,
        grid=grid,
        in_specs=[
            pl.BlockSpec((rb, 128), lambda i: (i, 0)),
            pl.BlockSpec((128, 256), lambda i: (0, 0)),
            pl.BlockSpec((1, 256), lambda i: (0, 0)),
        ],
        out_specs=[pl.BlockSpec((rb, 128), lambda i: (i, 0))] * 2,
        out_shape=[jax.ShapeDtypeStruct((np_, 128), F32)] * 2,
    )(xp, w, b)


def _invdeg(da, db, np_, rb):
    grid = (np_ // rb,)
    return pl.pallas_call(
        _invdeg_body,
        grid=grid,
        in_specs=[pl.BlockSpec((rb, 128), lambda i: (i, 0))] * 2,
        out_specs=pl.BlockSpec((rb, 1), lambda i: (i, 0)),
        out_shape=jax.ShapeDtypeStruct((np_, 1), F32),
    )(da, db)


def _mm_mid(z, pa, pb, iv, w, b, np_, rb):
    grid = (np_ // rb,)
    return pl.pallas_call(
        _mm_mid_body,
        grid=grid,
        in_specs=[
            pl.BlockSpec((rb, 128), lambda i: (i, 0)),
            pl.BlockSpec((rb, 128), lambda i: (i, 0)),
            pl.BlockSpec((rb, 128), lambda i: (i, 0)),
            pl.BlockSpec((rb, 1), lambda i: (i, 0)),
            pl.BlockSpec((128, 256), lambda i: (0, 0)),
            pl.BlockSpec((1, 256), lambda i: (0, 0)),
        ],
        out_specs=[pl.BlockSpec((rb, 128), lambda i: (i, 0))] * 2,
        out_shape=[jax.ShapeDtypeStruct((np_, 128), F32)] * 2,
    )(z, pa, pb, iv, w, b)


def _classify(zt, pat, pbt, dat, dbt, wc, bc, b_, c_):
    return pl.pallas_call(
        _classify_body,
        in_specs=[
            pl.BlockSpec((b_, 128), lambda: (0, 0)),
            pl.BlockSpec((b_, 128), lambda: (0, 0)),
            pl.BlockSpec((b_, 128), lambda: (0, 0)),
            pl.BlockSpec((b_, 128), lambda: (0, 0)),
            pl.BlockSpec((b_, 128), lambda: (0, 0)),
            pl.BlockSpec((128, c_), lambda: (0, 0)),
            pl.BlockSpec((1, c_), lambda: (0, 0)),
        ],
        out_specs=[
            pl.BlockSpec((b_, c_), lambda: (0, 0)),
            pl.BlockSpec((b_, 128), lambda: (0, 0)),
        ],
        out_shape=[
            jax.ShapeDtypeStruct((b_, c_), F32),
            jax.ShapeDtypeStruct((b_, 128), F32),
        ],
    )(zt, pat, pbt, dat, dbt, wc, bc)


def _make_agg(np_, nchunks):
    zr = np_ // NS
    mesh = plsc.VectorSubcoreMesh(core_axis_name="c", subcore_axis_name="s")

    scratch = [
        pltpu.VMEM((2, CHUNK), jnp.int32),
        pltpu.VMEM((2, CHUNK), jnp.int32),
        pltpu.VMEM((2, CHUNK, 128), F32),
        pltpu.VMEM_SHARED((np_, 128), F32),
        pltpu.SemaphoreType.DMA((2,)),
        pltpu.SemaphoreType.DMA((2,)),
    ]

    def body(y_hbm, srcb, dstb, zrow, out, src_v, dst_v, rows_v, acc,
             gsem, ssem):
        c = lax.axis_index("c")
        s = lax.axis_index("s")
        wid = c * NS + s
        pltpu.sync_copy(zrow, acc.at[pl.ds(s * zr, zr)])
        plsc.subcore_barrier()

        pltpu.sync_copy(srcb.at[wid, 0], src_v.at[0])
        pltpu.sync_copy(dstb.at[wid, 0], dst_v.at[0])
        pltpu.async_copy(y_hbm.at[src_v.at[0]], rows_v.at[0], gsem.at[0])

        def step(r, carry):
            bi = lax.rem(r, 2)
            nb = lax.rem(r + 1, 2)

            @pl.when(r + 1 < nchunks)
            def _():
                @pl.when(r >= 1)
                def _():
                    pltpu.make_async_copy(
                        rows_v.at[nb], acc.at[dst_v.at[nb]],
                        ssem.at[nb]).wait()
                pltpu.sync_copy(srcb.at[wid, r + 1], src_v.at[nb])
                pltpu.sync_copy(dstb.at[wid, r + 1], dst_v.at[nb])
                pltpu.async_copy(y_hbm.at[src_v.at[nb]], rows_v.at[nb],
                                 gsem.at[nb])

            pltpu.make_async_copy(y_hbm.at[src_v.at[bi]], rows_v.at[bi],
                                  gsem.at[bi]).wait()
            pltpu.async_copy(rows_v.at[bi], acc.at[dst_v.at[bi]],
                             ssem.at[bi], add=True)
            return carry

        lax.fori_loop(0, nchunks, step, 0)
        if nchunks >= 2:
            b2 = (nchunks - 2) % 2
            pltpu.make_async_copy(rows_v.at[b2], acc.at[dst_v.at[b2]],
                                  ssem.at[b2]).wait()
        b1 = (nchunks - 1) % 2
        pltpu.make_async_copy(rows_v.at[b1], acc.at[dst_v.at[b1]],
                              ssem.at[b1]).wait()
        plsc.subcore_barrier()
        pltpu.sync_copy(acc.at[pl.ds(s * zr, zr)],
                        out.at[c, pl.ds(s * zr, zr)])

    return pl.kernel(
        body,
        mesh=mesh,
        out_type=jax.ShapeDtypeStruct((NC, np_, 128), F32),
        scratch_types=scratch,
    )


def _make_deg(np_, nchunks):
    zr = np_ // NS
    mesh = plsc.VectorSubcoreMesh(core_axis_name="c", subcore_axis_name="s")

    scratch = [
        pltpu.VMEM((2, CHUNK), jnp.int32),
        pltpu.VMEM((CHUNK, 128), F32),
        pltpu.VMEM_SHARED((np_, 128), F32),
    ]

    def body(dstb, zrow, ones_h, out, dst_v, ones_v, acc):
        c = lax.axis_index("c")
        s = lax.axis_index("s")
        wid = c * NS + s
        pltpu.sync_copy(zrow, acc.at[pl.ds(s * zr, zr)])
        pltpu.sync_copy(ones_h, ones_v)
        plsc.subcore_barrier()

        pltpu.sync_copy(dstb.at[wid, 0], dst_v.at[0])

        def step(r, carry):
            bi = lax.rem(r, 2)
            nb = lax.rem(r + 1, 2)

            @pl.when(r + 1 < nchunks)
            def _():
                pltpu.sync_copy(dstb.at[wid, r + 1], dst_v.at[nb])

            pltpu.sync_copy(ones_v, acc.at[dst_v.at[bi]], add=True)
            return carry

        lax.fori_loop(0, nchunks, step, 0)
        plsc.subcore_barrier()
        pltpu.sync_copy(acc.at[pl.ds(s * zr, zr)],
                        out.at[c, pl.ds(s * zr, zr)])

    return pl.kernel(
        body,
        mesh=mesh,
        out_type=jax.ShapeDtypeStruct((NC, np_, 128), F32),
        scratch_types=scratch,
    )


def _make_target_gather(np_, b_, ntab):
    per = b_ // NW
    mesh = plsc.VectorSubcoreMesh(core_axis_name="c", subcore_axis_name="s")

    scratch = [
        pltpu.VMEM((per,), jnp.int32),
        pltpu.VMEM((per, 128), F32),
        pltpu.SemaphoreType.DMA,
    ]

    def body(*refs):
        tables = refs[:ntab]
        tgt_hbm, out, tgt_v, rows_v, sem = refs[ntab:]
        c = lax.axis_index("c")
        s = lax.axis_index("s")
        wid = c * NS + s
        pltpu.sync_copy(tgt_hbm.at[pl.ds(wid * per, per)], tgt_v)
        for k, tab in enumerate(tables):
            pltpu.async_copy(tab.at[tgt_v], rows_v, sem).wait()
            pltpu.sync_copy(rows_v, out.at[k, pl.ds(wid * per, per)])

    return pl.kernel(
        body,
        mesh=mesh,
        out_type=jax.ShapeDtypeStruct((ntab, b_, 128), F32),
        scratch_types=scratch,
    )


def kernel(feat, edge_index, target, Ws0, Wn0, b0, Ws1, Wn1, b1,
           Ws2, Wn2, b2, Wc, bc):
    n, d = feat.shape
    e = edge_index.shape[1]
    b_ = target.shape[0]
    c_ = Wc.shape[1]

    rb = 512
    np_ = ((n + rb - 1) // rb) * rb
    if np_ == n:
        np_ += rb
    nchunks = -(-e // (NW * CHUNK))
    e_pad = NW * CHUNK * nchunks

    src = edge_index[0].astype(jnp.int32)
    dst = edge_index[1].astype(jnp.int32)
    pad = e_pad - e
    dummy_src = jnp.arange(pad, dtype=jnp.int32) % jnp.int32(n)
    srcb = jnp.concatenate([src, dummy_src])
    dummy = n + (jnp.arange(pad, dtype=jnp.int32) % jnp.int32(np_ - n))
    dstb = jnp.concatenate([dst, dummy])
    srcb = srcb.reshape(NW, nchunks, CHUNK)
    dstb = dstb.reshape(NW, nchunks, CHUNK)
    featp = jnp.pad(feat, ((0, np_ - n), (0, 0)))

    w0 = jnp.concatenate([Ws0, Wn0], axis=1)
    w1 = jnp.concatenate([Ws1, Wn1], axis=1)
    w2 = jnp.concatenate([Ws2, Wn2], axis=1)
    zeros_h = jnp.zeros((128,), F32)
    bf0 = jnp.concatenate([b0, zeros_h]).reshape(1, 256)
    bf1 = jnp.concatenate([b1, zeros_h]).reshape(1, 256)
    bf2 = jnp.concatenate([b2, zeros_h]).reshape(1, 256)
    bcr = bc.reshape(1, c_)

    zr = np_ // NS
    zrow = jnp.zeros((zr, 128), F32)
    ones_rows = jnp.ones((CHUNK, 128), F32)

    agg = _make_agg(np_, nchunks)
    degk = _make_deg(np_, nchunks)
    tgather = _make_target_gather(np_, b_, 5)

    degp = degk(dstb, zrow, ones_rows)
    da, db = degp[0], degp[1]
    iv = _invdeg(da, db, np_, rb)
    z0, y0 = _mm_first(featp, w0, bf0, np_, rb)
    p0 = agg(y0, srcb, dstb, zrow)
    z1, y1 = _mm_mid(z0, p0[0], p0[1], iv, w1, bf1, np_, rb)
    p1 = agg(y1, srcb, dstb, zrow)
    z2, y2 = _mm_mid(z1, p1[0], p1[1], iv, w2, bf2, np_, rb)
    p2 = agg(y2, srcb, dstb, zrow)
    g = tgather(z2, p2[0], p2[1], da, db, target.astype(jnp.int32))
    pred, emb = _classify(g[0], g[1], g[2], g[3], g[4], Wc, bcr, b_, c_)
    return (pred, emb)
